# Initial kernel scaffold; baseline (speedup 1.0000x reference)
import jax
import jax.numpy as jnp
from jax.experimental import pallas as pl


def kernel(array, cell_area_at_node, links_at_node, link_dirs_at_node, status_at_node):
    raise NotImplementedError("write your pallas kernel here")


# speedup vs baseline: 37.6103x; 37.6103x over previous
import functools

import jax
import jax.numpy as jnp
from jax import lax
from jax.experimental import pallas as pl
from jax.experimental.pallas import tpu as pltpu
from jax.experimental.pallas import tpu_sc as plsc

NC = 2
NS = 16
NW = NC * NS
LANES = 16
K = 4

N_NODES = 100000
N_PER_W = 3136
N_PAD = NW * N_PER_W
CHUNKS = N_PER_W // LANES
IDX_PER_W = K * N_PER_W


def _sc_body(array_hbm, idx_hbm, dirs_hbm, status_hbm, area_hbm, out_hbm,
             idx_v, gath_v, dirs_v, status_v, area_v, out_v, sem):
    wid = lax.axis_index("s") * NC + lax.axis_index("c")
    pltpu.sync_copy(idx_hbm.at[wid], idx_v)
    pltpu.sync_copy(dirs_hbm.at[wid], dirs_v)
    pltpu.sync_copy(status_hbm.at[wid], status_v)
    pltpu.sync_copy(area_hbm.at[wid], area_v)
    pltpu.async_copy(array_hbm.at[idx_v], gath_v, sem).wait()

    def chunk(c, carry):
        off = c * LANES
        acc = jnp.zeros((LANES,), jnp.float32)
        for j in range(K):
            g = gath_v[pl.ds(j * N_PER_W + off, LANES)]
            d = dirs_v[pl.ds(j * N_PER_W + off, LANES)]
            acc = acc + d * g
        st = status_v[pl.ds(off, LANES)]
        ar = area_v[pl.ds(off, LANES)]
        out_v[pl.ds(off, LANES)] = jnp.where(st == 0, acc / ar, 0.0)
        return carry

    lax.fori_loop(0, CHUNKS, chunk, 0)
    pltpu.sync_copy(out_v, out_hbm.at[wid])


@jax.jit
def _flux_div_sc(array, idx_w, dirs_w, status_w, area_w):
    mesh = plsc.VectorSubcoreMesh(core_axis_name="c", subcore_axis_name="s")
    run = pl.kernel(
        _sc_body,
        out_type=jax.ShapeDtypeStruct((NW, N_PER_W), jnp.float32),
        mesh=mesh,
        scratch_types=[
            pltpu.VMEM((IDX_PER_W,), jnp.int32),
            pltpu.VMEM((IDX_PER_W,), jnp.float32),
            pltpu.VMEM((IDX_PER_W,), jnp.float32),
            pltpu.VMEM((N_PER_W,), jnp.int32),
            pltpu.VMEM((N_PER_W,), jnp.float32),
            pltpu.VMEM((N_PER_W,), jnp.float32),
            pltpu.SemaphoreType.DMA,
        ],
    )
    return run(array, idx_w, dirs_w, status_w, area_w)


def kernel(array, cell_area_at_node, links_at_node, link_dirs_at_node, status_at_node):
    pad = N_PAD - N_NODES
    links_p = jnp.pad(links_at_node, ((0, pad), (0, 0)))
    dirs_p = jnp.pad(link_dirs_at_node, ((0, pad), (0, 0)))
    status_p = jnp.pad(status_at_node, (0, pad), constant_values=1)
    area_p = jnp.pad(cell_area_at_node, (0, pad), constant_values=1.0)
    idx_w = (links_p.T.reshape(K, NW, N_PER_W)
             .transpose(1, 0, 2).reshape(NW, IDX_PER_W))
    dirs_w = (dirs_p.astype(jnp.float32).T.reshape(K, NW, N_PER_W)
              .transpose(1, 0, 2).reshape(NW, IDX_PER_W))
    status_w = status_p.reshape(NW, N_PER_W)
    area_w = area_p.reshape(NW, N_PER_W)
    out = _flux_div_sc(array, idx_w, dirs_w, status_w, area_w)
    return out.reshape(N_PAD)[:N_NODES]
